# baseline (speedup 1.0000x reference)
import functools

import jax
import jax.numpy as jnp
from jax import lax
from jax.experimental import pallas as pl
from jax.experimental.pallas import tpu as pltpu
from jax.experimental.pallas import tpu_sc as plsc

N = 10000
E = 320000
D = 128
H = 8
DH = 16
HD = H * DH
HH = HD // 2
HOP = 3
ALPHA = 0.15
NEG = 0.2
C = 40

NC = 1
NS = 16
NW = NC * NS
EPW = E // NW
CH = 80
NCH = EPW // CH
NP = 10240
RPT = NP // NS
ZR = RPT // 2

BN = 400
NBLK = N // BN
_MESH = plsc.VectorSubcoreMesh(
    core_axis_name="c", subcore_axis_name="s", num_cores=NC, num_subcores=NS)


def _make_phase_a():
  @functools.partial(
      pl.kernel,
      out_type=(
          jax.ShapeDtypeStruct((NW, NCH, CH, 16), jnp.float32),
          jax.ShapeDtypeStruct((NC, NP, 16), jnp.float32),
      ),
      mesh=_MESH,
      name="gdt_phase_a",
      compiler_params=pltpu.CompilerParams(use_tc_tiling_on_sc=False),
      scratch_types=[
          pltpu.VMEM((NCH, CH), jnp.int32),
          pltpu.VMEM((NCH, CH), jnp.int32),
          pltpu.VMEM((CH, 16), jnp.float32),
          pltpu.VMEM((CH, 16), jnp.float32),
          pltpu.VMEM((CH, 16), jnp.float32),
          pltpu.VMEM((CH, 16), jnp.float32),
          pltpu.VMEM((CH, 16), jnp.float32),
          pltpu.VMEM((RPT, 16), jnp.float32),
          pltpu.VMEM_SHARED((NP, 16), jnp.float32),
          pltpu.SemaphoreType.DMA,
          pltpu.SemaphoreType.DMA,
          pltpu.SemaphoreType.DMA,
          pltpu.SemaphoreType.DMA,
      ],
  )
  def k(el_h, er_h, edge_h, ex_o, sp_o, src_i, dst_i, abuf0, abuf1, bbuf0,
        bbuf1, exbuf, zb, sacc, asem0, asem1, bsem0, bsem1):
    c = lax.axis_index("c")
    s = lax.axis_index("s")
    wid = c * NS + s
    abufs = (abuf0, abuf1)
    bbufs = (bbuf0, bbuf1)
    asems = (asem0, asem1)
    bsems = (bsem0, bsem1)

    pltpu.sync_copy(edge_h.at[0, wid], src_i)
    pltpu.sync_copy(edge_h.at[1, wid], dst_i)

    def zloop(i, _):
      zb[i, :] = jnp.zeros((16,), jnp.float32)
      return 0
    lax.fori_loop(0, RPT, zloop, 0)
    pltpu.sync_copy(zb, sacc.at[pl.ds(s * RPT, RPT)])
    plsc.subcore_barrier()

    lane = lax.iota(jnp.int32, 16)
    maskb = lane < 8

    def issue(j, b):
      pltpu.async_copy(el_h.at[src_i.at[j]], abufs[b], asems[b])
      pltpu.async_copy(er_h.at[dst_i.at[j]], bbufs[b], bsems[b])

    def step(j, b, issue_next):
      pltpu.make_async_copy(el_h.at[src_i.at[j]], abufs[b], asems[b]).wait()
      pltpu.make_async_copy(er_h.at[dst_i.at[j]], bbufs[b], bsems[b]).wait()
      abuf = abufs[b]
      bbuf = bbufs[b]

      def edge(i, _):
        t = abuf[i, :] + bbuf[i, :]
        e = jnp.maximum(t, NEG * t)
        exbuf[i, :] = jnp.where(maskb, jnp.exp(e), 0.0)
        return 0
      lax.fori_loop(0, CH, edge, 0)

      pltpu.sync_copy(exbuf, ex_o.at[wid, j])
      pltpu.sync_copy(exbuf, sacc.at[dst_i.at[j]], add=True)
      if issue_next:
        issue(j + 2, b)

    issue(0, 0)
    issue(1, 1)

    def pair(t, _):
      step(2 * t, 0, True)
      step(2 * t + 1, 1, True)
      return 0
    lax.fori_loop(0, NCH // 2 - 1, pair, 0)
    step(NCH - 2, 0, False)
    step(NCH - 1, 1, False)

    plsc.subcore_barrier()
    pltpu.sync_copy(sacc.at[pl.ds(s * RPT, RPT)],
                    sp_o.at[c, pl.ds(s * RPT, RPT)])

  return k


_phase_a = _make_phase_a()


def _make_hop():
  @functools.partial(
      pl.kernel,
      out_type=jax.ShapeDtypeStruct((2, NP, 1, HH), jnp.float32),
      mesh=_MESH,
      name="gdt_hop",
      compiler_params=pltpu.CompilerParams(use_tc_tiling_on_sc=False),
      scratch_types=[
          pltpu.VMEM((NCH, CH), jnp.int32),
          pltpu.VMEM((NCH, CH), jnp.int32),
          pltpu.VMEM((CH, 1, HH), jnp.float32),
          pltpu.VMEM((CH, 1, HH), jnp.float32),
          pltpu.VMEM((CH, 1, HH), jnp.float32),
          pltpu.VMEM((CH, 1, HH), jnp.float32),
          pltpu.VMEM((CH, 16), jnp.float32),
          pltpu.VMEM((CH, 16), jnp.float32),
          pltpu.VMEM((ZR, 1, HH), jnp.float32),
          pltpu.VMEM_SHARED((NP, 1, HH), jnp.float32),
          pltpu.SemaphoreType.DMA,
          pltpu.SemaphoreType.DMA,
          pltpu.SemaphoreType.DMA,
          pltpu.SemaphoreType.DMA,
          pltpu.SemaphoreType.DMA,
          pltpu.SemaphoreType.DMA,
      ],
  )
  def k(fa_h, fb_h, ex_h, edge_h, agg_o, src_i, dst_i, gbuf0, gbuf1, sbuf0,
        sbuf1, exb0, exb1, zb, acc, gsem0, gsem1, esem0, esem1, ssem0, ssem1):
    c = lax.axis_index("c")
    s = lax.axis_index("s")
    wid = c * NS + s
    gbufs = (gbuf0, gbuf1)
    sbufs = (sbuf0, sbuf1)
    exbs = (exb0, exb1)
    gsems = (gsem0, gsem1)
    esems = (esem0, esem1)
    ssems = (ssem0, ssem1)

    pltpu.sync_copy(edge_h.at[0, wid], src_i)
    pltpu.sync_copy(edge_h.at[1, wid], dst_i)

    def zloop(i, _):
      for q in range(HH // 16):
        zb[i, 0, pl.ds(q * 16, 16)] = jnp.zeros((16,), jnp.float32)
      return 0
    lax.fori_loop(0, ZR, zloop, 0)

    splat_idx = [jnp.full((16,), hh, jnp.int32) for hh in range(H)]

    for p in range(2):
      f_h = (fa_h, fb_h)[p]
      pltpu.sync_copy(zb, acc.at[pl.ds(s * RPT, ZR)])
      pltpu.sync_copy(zb, acc.at[pl.ds(s * RPT + ZR, ZR)])
      plsc.subcore_barrier()

      def issue(j, b):
        pltpu.async_copy(f_h.at[src_i.at[j]], gbufs[b], gsems[b])
        pltpu.async_copy(ex_h.at[wid, j], exbs[b], esems[b])

      def step(j, b, issue_next, drain_scatter):
        pltpu.make_async_copy(f_h.at[src_i.at[j]], gbufs[b], gsems[b]).wait()
        pltpu.make_async_copy(ex_h.at[wid, j], exbs[b], esems[b]).wait()
        if drain_scatter:
          pltpu.make_async_copy(sbufs[b], acc.at[dst_i.at[j]],
                                ssems[b]).wait()

        gbuf = gbufs[b]
        sbuf = sbufs[b]
        exb = exbs[b]

        def edge(i, _):
          exrow = exb[i, :]
          for hq in range(HH // DH):
            splat = jnp.take_along_axis(
                exrow, splat_idx[p * (HH // DH) + hq], axis=0)
            v = gbuf[i, 0, pl.ds(hq * DH, 16)]
            sbuf[i, 0, pl.ds(hq * DH, 16)] = v * splat
          return 0
        lax.fori_loop(0, CH, edge, 0, unroll=2)

        if issue_next:
          issue(j + 2, b)
        pltpu.async_copy(sbuf, acc.at[dst_i.at[j]], ssems[b], add=True)

      issue(0, 0)
      issue(1, 1)
      step(0, 0, True, False)
      step(1, 1, True, False)

      def pair(t, _):
        step(2 * t, 0, True, True)
        step(2 * t + 1, 1, True, True)
        return 0
      lax.fori_loop(1, NCH // 2 - 1, pair, 0)
      step(NCH - 2, 0, False, True)
      step(NCH - 1, 1, False, True)
      pltpu.make_async_copy(sbufs[0], acc.at[dst_i.at[0]], ssems[0]).wait()
      pltpu.make_async_copy(sbufs[1], acc.at[dst_i.at[0]], ssems[1]).wait()

      plsc.subcore_barrier()
      pltpu.sync_copy(acc.at[pl.ds(s * RPT, RPT)],
                      agg_o.at[p, pl.ds(s * RPT, RPT)])
      if p == 0:
        plsc.subcore_barrier()

  return k


_hop = _make_hop()


def _dense_pre(h, W, P, Wr):
  def body(h_ref, w_ref, p_ref, wr_ref, fa_ref, fb_ref, elb_ref, hres_ref):
    hb = h_ref[...]
    f = jnp.dot(hb, w_ref[...], preferred_element_type=jnp.float32)
    fa_ref[...] = f[:, :HH]
    fb_ref[...] = f[:, HH:]
    elb_ref[...] = jnp.dot(f, p_ref[...], preferred_element_type=jnp.float32)
    hres_ref[...] = jnp.dot(hb, wr_ref[...], preferred_element_type=jnp.float32)

  full = pl.BlockSpec((D, HD), lambda i: (0, 0))
  row = pl.BlockSpec((BN, HD), lambda i: (i, 0))
  half = pl.BlockSpec((BN, HH), lambda i: (i, 0))
  return pl.pallas_call(
      body,
      grid=(NBLK,),
      in_specs=[pl.BlockSpec((BN, D), lambda i: (i, 0)), full, full, full],
      out_specs=[half, half, row, row],
      out_shape=[
          jax.ShapeDtypeStruct((N, HH), jnp.float32),
          jax.ShapeDtypeStruct((N, HH), jnp.float32),
          jax.ShapeDtypeStruct((N, HD), jnp.float32),
          jax.ShapeDtypeStruct((N, HD), jnp.float32),
      ],
  )(h, W, P, Wr)


def _inv_full(s_part, R):
  def body(s_ref, r_ref, oa_ref, ob_ref):
    inv = 1.0 / (s_ref[0] + 1e-16)
    full = jnp.dot(inv, r_ref[...], preferred_element_type=jnp.float32)
    oa_ref[...] = full[:, :HH]
    ob_ref[...] = full[:, HH:]

  half = pl.BlockSpec((BN, HH), lambda i: (i, 0))
  return pl.pallas_call(
      body,
      grid=(NBLK,),
      in_specs=[
          pl.BlockSpec((NC, BN, 16), lambda i: (0, i, 0)),
          pl.BlockSpec((16, HD), lambda i: (0, 0)),
      ],
      out_specs=[half, half],
      out_shape=[jax.ShapeDtypeStruct((N, HH), jnp.float32)] * 2,
  )(s_part, R)


def _combine(agg, inva, invb, fa0, fb0, hres):
  last = hres is not None

  def body(a_ref, ia_ref, ib_ref, fa0_ref, fb0_ref, *rest):
    if last:
      hres_ref, o_ref = rest
    else:
      oa_ref, ob_ref = rest
    ta = (1.0 - ALPHA) * a_ref[0, :, 0, :] * ia_ref[...] + ALPHA * fa0_ref[...]
    tb = (1.0 - ALPHA) * a_ref[1, :, 0, :] * ib_ref[...] + ALPHA * fb0_ref[...]
    if last:
      t = jnp.concatenate([ta, tb], axis=1) + hres_ref[...]
      o_ref[...] = jnp.where(t > 0, t, jnp.exp(jnp.minimum(t, 0.0)) - 1.0)
    else:
      oa_ref[...] = ta
      ob_ref[...] = tb

  half = pl.BlockSpec((BN, HH), lambda i: (i, 0))
  row = pl.BlockSpec((BN, HD), lambda i: (i, 0))
  in_specs = [pl.BlockSpec((2, BN, 1, HH), lambda i: (0, i, 0, 0)),
              half, half, half, half]
  args = [agg, inva, invb, fa0, fb0]
  if last:
    in_specs.append(row)
    args.append(hres)
    out_specs = row
    out_shape = jax.ShapeDtypeStruct((N, HD), jnp.float32)
  else:
    out_specs = [half, half]
    out_shape = [jax.ShapeDtypeStruct((N, HH), jnp.float32)] * 2
  return pl.pallas_call(
      body,
      grid=(NBLK,),
      in_specs=in_specs,
      out_specs=out_specs,
      out_shape=out_shape,
  )(*args)


def _classify(h, Wcp, bcp):
  def body(h_ref, w_ref, b_ref, o_ref):
    o_ref[...] = (jnp.dot(h_ref[...], w_ref[...],
                          preferred_element_type=jnp.float32) + b_ref[...])

  return pl.pallas_call(
      body,
      grid=(NBLK,),
      in_specs=[
          pl.BlockSpec((BN, HD), lambda i: (i, 0)),
          pl.BlockSpec((HD, HD), lambda i: (0, 0)),
          pl.BlockSpec((1, HD), lambda i: (0, 0)),
      ],
      out_specs=pl.BlockSpec((BN, HD), lambda i: (i, 0)),
      out_shape=jax.ShapeDtypeStruct((N, HD), jnp.float32),
  )(h, Wcp, bcp)


def kernel(x, edge_index, W1, al1, ar1, Wr1, W2, al2, ar2, Wr2, Wc, bc):
  ei = edge_index.astype(jnp.int32).reshape(2, NW, NCH, CH)
  k128 = jnp.arange(HD)

  def proj(al, ar):
    alp = jnp.zeros((HD, 16), jnp.float32).at[k128, k128 // DH].set(
        al.reshape(HD))
    arp = jnp.zeros((HD, 16), jnp.float32).at[k128, k128 // DH].set(
        ar.reshape(HD))
    return jnp.concatenate(
        [alp, arp, jnp.zeros((HD, HD - 32), jnp.float32)], axis=1)

  P1 = proj(al1, ar1)
  P2 = proj(al2, ar2)
  R = jnp.zeros((16, HD), jnp.float32).at[k128 // DH, k128].set(1.0)
  Wcp = jnp.zeros((HD, HD), jnp.float32).at[:, :C].set(Wc)
  bcp = jnp.zeros((1, HD), jnp.float32).at[0, :C].set(bc)

  h = x
  for (W, P, Wr) in ((W1, P1, Wr1), (W2, P2, Wr2)):
    fa0, fb0, elboth, hres = _dense_pre(h, W, P, Wr)
    el16 = elboth[:, 0:16]
    er16 = elboth[:, 16:32]
    ex, s_part = _phase_a(el16, er16, ei)
    inva, invb = _inv_full(s_part, R)
    fa, fb = fa0, fb0
    for hop in range(HOP):
      agg = _hop(fa.reshape(N, 1, HH), fb.reshape(N, 1, HH), ex, ei)
      out = _combine(agg, inva, invb, fa0, fb0,
                     hres if hop == HOP - 1 else None)
      if hop == HOP - 1:
        h = out
      else:
        fa, fb = out

  logits = _classify(h, Wcp, bcp)
  return logits[:, :C]


# speedup vs baseline: 1.5563x; 1.5563x over previous
import functools

import jax
import jax.numpy as jnp
from jax import lax
from jax.experimental import pallas as pl
from jax.experimental.pallas import tpu as pltpu
from jax.experimental.pallas import tpu_sc as plsc

N = 10000
E = 320000
D = 128
H = 8
DH = 16
HD = H * DH
HH = HD // 2
HOP = 3
ALPHA = 0.15
NEG = 0.2
C = 40

NC = 1
NS = 16
NW = NC * NS
EPW = E // NW
CH = 80
NCH = EPW // CH
NP = 10240
RPT = NP // NS
ZR = RPT // 2

BN = 400
NBLK = N // BN
_MESH = plsc.VectorSubcoreMesh(
    core_axis_name="c", subcore_axis_name="s", num_cores=NC, num_subcores=NS)


def _make_phase_a():
  @functools.partial(
      pl.kernel,
      out_type=(
          jax.ShapeDtypeStruct((NW, NCH, CH, 16), jnp.float32),
          jax.ShapeDtypeStruct((NC, NP, 16), jnp.float32),
      ),
      mesh=_MESH,
      name="gdt_phase_a",
      compiler_params=pltpu.CompilerParams(use_tc_tiling_on_sc=False),
      scratch_types=[
          pltpu.VMEM((NCH, CH), jnp.int32),
          pltpu.VMEM((NCH, CH), jnp.int32),
          pltpu.VMEM((CH, 16), jnp.float32),
          pltpu.VMEM((CH, 16), jnp.float32),
          pltpu.VMEM((CH, 16), jnp.float32),
          pltpu.VMEM((CH, 16), jnp.float32),
          pltpu.VMEM((CH, 16), jnp.float32),
          pltpu.VMEM((RPT, 16), jnp.float32),
          pltpu.VMEM_SHARED((NP, 16), jnp.float32),
          pltpu.SemaphoreType.DMA,
          pltpu.SemaphoreType.DMA,
          pltpu.SemaphoreType.DMA,
          pltpu.SemaphoreType.DMA,
      ],
  )
  def k(el_h, er_h, edge_h, ex_o, sp_o, src_i, dst_i, abuf0, abuf1, bbuf0,
        bbuf1, exbuf, zb, sacc, asem0, asem1, bsem0, bsem1):
    c = lax.axis_index("c")
    s = lax.axis_index("s")
    wid = c * NS + s
    abufs = (abuf0, abuf1)
    bbufs = (bbuf0, bbuf1)
    asems = (asem0, asem1)
    bsems = (bsem0, bsem1)

    pltpu.sync_copy(edge_h.at[0, wid], src_i)
    pltpu.sync_copy(edge_h.at[1, wid], dst_i)

    def zloop(i, _):
      zb[i, :] = jnp.zeros((16,), jnp.float32)
      return 0
    lax.fori_loop(0, RPT, zloop, 0)
    pltpu.sync_copy(zb, sacc.at[pl.ds(s * RPT, RPT)])
    plsc.subcore_barrier()

    lane = lax.iota(jnp.int32, 16)
    maskb = lane < 8

    def issue(j, b):
      pltpu.async_copy(el_h.at[src_i.at[j]], abufs[b], asems[b])
      pltpu.async_copy(er_h.at[dst_i.at[j]], bbufs[b], bsems[b])

    def step(j, b, issue_next):
      pltpu.make_async_copy(el_h.at[src_i.at[j]], abufs[b], asems[b]).wait()
      pltpu.make_async_copy(er_h.at[dst_i.at[j]], bbufs[b], bsems[b]).wait()
      abuf = abufs[b]
      bbuf = bbufs[b]

      def edge(i, _):
        t = abuf[i, :] + bbuf[i, :]
        e = jnp.maximum(t, NEG * t)
        exbuf[i, :] = jnp.where(maskb, jnp.exp(e), 0.0)
        return 0
      lax.fori_loop(0, CH, edge, 0)

      pltpu.sync_copy(exbuf, ex_o.at[wid, j])
      pltpu.sync_copy(exbuf, sacc.at[dst_i.at[j]], add=True)
      if issue_next:
        issue(j + 2, b)

    issue(0, 0)
    issue(1, 1)

    def pair(t, _):
      step(2 * t, 0, True)
      step(2 * t + 1, 1, True)
      return 0
    lax.fori_loop(0, NCH // 2 - 1, pair, 0)
    step(NCH - 2, 0, False)
    step(NCH - 1, 1, False)

    plsc.subcore_barrier()
    pltpu.sync_copy(sacc.at[pl.ds(s * RPT, RPT)],
                    sp_o.at[c, pl.ds(s * RPT, RPT)])

  return k


_phase_a = _make_phase_a()


def _make_hop():
  @functools.partial(
      pl.kernel,
      out_type=jax.ShapeDtypeStruct((2, NP, 1, HH), jnp.float32),
      mesh=_MESH,
      name="gdt_hop",
      compiler_params=pltpu.CompilerParams(use_tc_tiling_on_sc=False),
      scratch_types=[
          pltpu.VMEM((NCH, CH), jnp.int32),
          pltpu.VMEM((NCH, CH), jnp.int32),
          pltpu.VMEM((CH, 1, HH), jnp.float32),
          pltpu.VMEM((CH, 1, HH), jnp.float32),
          pltpu.VMEM((CH, 16), jnp.float32),
          pltpu.VMEM((CH, 16), jnp.float32),
          pltpu.VMEM((ZR, 1, HH), jnp.float32),
          pltpu.VMEM_SHARED((NP, 1, HH), jnp.float32),
          pltpu.SemaphoreType.DMA,
          pltpu.SemaphoreType.DMA,
          pltpu.SemaphoreType.DMA,
          pltpu.SemaphoreType.DMA,
      ],
  )
  def k(fa_h, fb_h, ex_h, edge_h, agg_o, src_i, dst_i, frow0, frow1, exb0,
        exb1, zb, acc, gsem0, gsem1, esem0, esem1):
    c = lax.axis_index("c")
    s = lax.axis_index("s")
    wid = c * NS + s
    frows = (frow0, frow1)
    exbs = (exb0, exb1)
    gsems = (gsem0, gsem1)
    esems = (esem0, esem1)

    pltpu.sync_copy(edge_h.at[0, wid], src_i)
    pltpu.sync_copy(edge_h.at[1, wid], dst_i)

    def zloop(i, _):
      for q in range(HH // 16):
        zb[i, 0, pl.ds(q * 16, 16)] = jnp.zeros((16,), jnp.float32)
      return 0
    lax.fori_loop(0, ZR, zloop, 0)

    splat_idx = [jnp.full((16,), hh, jnp.int32) for hh in range(H)]

    for p in range(2):
      f_h = (fa_h, fb_h)[p]
      pltpu.sync_copy(zb, acc.at[pl.ds(s * RPT, ZR)])
      pltpu.sync_copy(zb, acc.at[pl.ds(s * RPT + ZR, ZR)])
      plsc.subcore_barrier()

      def issue(j, b):
        pltpu.async_copy(f_h.at[src_i.at[j]], frows[b], gsems[b])
        pltpu.async_copy(ex_h.at[wid, j], exbs[b], esems[b])

      def step(j, b, issue_next):
        pltpu.make_async_copy(f_h.at[src_i.at[j]], frows[b], gsems[b]).wait()
        pltpu.make_async_copy(ex_h.at[wid, j], exbs[b], esems[b]).wait()

        frow = frows[b]
        exb = exbs[b]

        def edge(i, _):
          exrow = exb[i, :]
          for hq in range(HH // DH):
            splat = jnp.take_along_axis(
                exrow, splat_idx[p * (HH // DH) + hq], axis=0)
            v = frow[i, 0, pl.ds(hq * DH, 16)]
            frow[i, 0, pl.ds(hq * DH, 16)] = v * splat
          return 0
        lax.fori_loop(0, CH, edge, 0)

        pltpu.sync_copy(frow, acc.at[dst_i.at[j]], add=True)
        if issue_next:
          issue(j + 2, b)

      issue(0, 0)
      issue(1, 1)

      def pair(t, _):
        step(2 * t, 0, True)
        step(2 * t + 1, 1, True)
        return 0
      lax.fori_loop(0, NCH // 2 - 1, pair, 0)
      step(NCH - 2, 0, False)
      step(NCH - 1, 1, False)

      plsc.subcore_barrier()
      pltpu.sync_copy(acc.at[pl.ds(s * RPT, RPT)],
                      agg_o.at[p, pl.ds(s * RPT, RPT)])
      if p == 0:
        plsc.subcore_barrier()

  return k


_hop = _make_hop()


def _dense_pre(h, W, P, Wr):
  def body(h_ref, w_ref, p_ref, wr_ref, fa_ref, fb_ref, elb_ref, hres_ref):
    hb = h_ref[...]
    f = jnp.dot(hb, w_ref[...], preferred_element_type=jnp.float32)
    fa_ref[...] = f[:, :HH]
    fb_ref[...] = f[:, HH:]
    elb_ref[...] = jnp.dot(f, p_ref[...], preferred_element_type=jnp.float32)
    hres_ref[...] = jnp.dot(hb, wr_ref[...], preferred_element_type=jnp.float32)

  full = pl.BlockSpec((D, HD), lambda i: (0, 0))
  row = pl.BlockSpec((BN, HD), lambda i: (i, 0))
  half = pl.BlockSpec((BN, HH), lambda i: (i, 0))
  return pl.pallas_call(
      body,
      grid=(NBLK,),
      in_specs=[pl.BlockSpec((BN, D), lambda i: (i, 0)), full, full, full],
      out_specs=[half, half, row, row],
      out_shape=[
          jax.ShapeDtypeStruct((N, HH), jnp.float32),
          jax.ShapeDtypeStruct((N, HH), jnp.float32),
          jax.ShapeDtypeStruct((N, HD), jnp.float32),
          jax.ShapeDtypeStruct((N, HD), jnp.float32),
      ],
  )(h, W, P, Wr)


def _inv_full(s_part, R):
  def body(s_ref, r_ref, oa_ref, ob_ref):
    inv = 1.0 / (s_ref[0] + 1e-16)
    full = jnp.dot(inv, r_ref[...], preferred_element_type=jnp.float32)
    oa_ref[...] = full[:, :HH]
    ob_ref[...] = full[:, HH:]

  half = pl.BlockSpec((BN, HH), lambda i: (i, 0))
  return pl.pallas_call(
      body,
      grid=(NBLK,),
      in_specs=[
          pl.BlockSpec((NC, BN, 16), lambda i: (0, i, 0)),
          pl.BlockSpec((16, HD), lambda i: (0, 0)),
      ],
      out_specs=[half, half],
      out_shape=[jax.ShapeDtypeStruct((N, HH), jnp.float32)] * 2,
  )(s_part, R)


def _combine(agg, inva, invb, fa0, fb0, hres):
  last = hres is not None

  def body(a_ref, ia_ref, ib_ref, fa0_ref, fb0_ref, *rest):
    if last:
      hres_ref, o_ref = rest
    else:
      oa_ref, ob_ref = rest
    ta = (1.0 - ALPHA) * a_ref[0, :, 0, :] * ia_ref[...] + ALPHA * fa0_ref[...]
    tb = (1.0 - ALPHA) * a_ref[1, :, 0, :] * ib_ref[...] + ALPHA * fb0_ref[...]
    if last:
      t = jnp.concatenate([ta, tb], axis=1) + hres_ref[...]
      o_ref[...] = jnp.where(t > 0, t, jnp.exp(jnp.minimum(t, 0.0)) - 1.0)
    else:
      oa_ref[...] = ta
      ob_ref[...] = tb

  half = pl.BlockSpec((BN, HH), lambda i: (i, 0))
  row = pl.BlockSpec((BN, HD), lambda i: (i, 0))
  in_specs = [pl.BlockSpec((2, BN, 1, HH), lambda i: (0, i, 0, 0)),
              half, half, half, half]
  args = [agg, inva, invb, fa0, fb0]
  if last:
    in_specs.append(row)
    args.append(hres)
    out_specs = row
    out_shape = jax.ShapeDtypeStruct((N, HD), jnp.float32)
  else:
    out_specs = [half, half]
    out_shape = [jax.ShapeDtypeStruct((N, HH), jnp.float32)] * 2
  return pl.pallas_call(
      body,
      grid=(NBLK,),
      in_specs=in_specs,
      out_specs=out_specs,
      out_shape=out_shape,
  )(*args)


def _classify(h, Wcp, bcp):
  def body(h_ref, w_ref, b_ref, o_ref):
    o_ref[...] = (jnp.dot(h_ref[...], w_ref[...],
                          preferred_element_type=jnp.float32) + b_ref[...])

  return pl.pallas_call(
      body,
      grid=(NBLK,),
      in_specs=[
          pl.BlockSpec((BN, HD), lambda i: (i, 0)),
          pl.BlockSpec((HD, HD), lambda i: (0, 0)),
          pl.BlockSpec((1, HD), lambda i: (0, 0)),
      ],
      out_specs=pl.BlockSpec((BN, HD), lambda i: (i, 0)),
      out_shape=jax.ShapeDtypeStruct((N, HD), jnp.float32),
  )(h, Wcp, bcp)


def kernel(x, edge_index, W1, al1, ar1, Wr1, W2, al2, ar2, Wr2, Wc, bc):
  ei = edge_index.astype(jnp.int32).reshape(2, NW, NCH, CH)
  k128 = jnp.arange(HD)

  def proj(al, ar):
    alp = jnp.zeros((HD, 16), jnp.float32).at[k128, k128 // DH].set(
        al.reshape(HD))
    arp = jnp.zeros((HD, 16), jnp.float32).at[k128, k128 // DH].set(
        ar.reshape(HD))
    return jnp.concatenate(
        [alp, arp, jnp.zeros((HD, HD - 32), jnp.float32)], axis=1)

  P1 = proj(al1, ar1)
  P2 = proj(al2, ar2)
  R = jnp.zeros((16, HD), jnp.float32).at[k128 // DH, k128].set(1.0)
  Wcp = jnp.zeros((HD, HD), jnp.float32).at[:, :C].set(Wc)
  bcp = jnp.zeros((1, HD), jnp.float32).at[0, :C].set(bc)

  h = x
  for (W, P, Wr) in ((W1, P1, Wr1), (W2, P2, Wr2)):
    fa0, fb0, elboth, hres = _dense_pre(h, W, P, Wr)
    el16 = elboth[:, 0:16]
    er16 = elboth[:, 16:32]
    ex, s_part = _phase_a(el16, er16, ei)
    inva, invb = _inv_full(s_part, R)
    fa, fb = fa0, fb0
    for hop in range(HOP):
      agg = _hop(fa.reshape(N, 1, HH), fb.reshape(N, 1, HH), ex, ei)
      out = _combine(agg, inva, invb, fa0, fb0,
                     hres if hop == HOP - 1 else None)
      if hop == HOP - 1:
        h = out
      else:
        fa, fb = out

  logits = _classify(h, Wcp, bcp)
  return logits[:, :C]
